# Initial kernel scaffold; baseline (speedup 1.0000x reference)
import functools

import jax
import jax.numpy as jnp
from jax import lax
from jax.experimental import pallas as pl
from jax.experimental.pallas import tpu as pltpu
from jax.experimental.pallas import tpu_sc as plsc

_NC = 2
_NS = 16
_NW = _NC * _NS
_CK = 80

_BE = 2560
_BN = 2000


def _ln(x, s, b):
    mu = jnp.mean(x, axis=-1, keepdims=True)
    xc = x - mu
    var = jnp.mean(xc * xc, axis=-1, keepdims=True)
    return xc * lax.rsqrt(var + 1e-5) * s + b


def _row_spec(br, c):
    return pl.BlockSpec((br, c), lambda i: (i, 0))


def _full_spec(shape):
    nd = len(shape)
    return pl.BlockSpec(shape, lambda i, _n=nd: (0,) * _n)


def _enc_node_body(x_ref, w1_ref, b1_ref, w2_ref, b2_ref, s_ref, bb_ref,
                   w1s_ref, w1r_ref, h_ref, hs_ref, hr_ref):
    t = jnp.maximum(x_ref[...] @ w1_ref[...] + b1_ref[...], 0.0)
    u = t @ w2_ref[...] + b2_ref[...]
    h = _ln(u, s_ref[...], bb_ref[...])
    h_ref[...] = h
    hs_ref[...] = h @ w1s_ref[...]
    hr_ref[...] = h @ w1r_ref[...]


def _enc_edge_body(x_ref, w1_ref, b1_ref, w2_ref, b2_ref, s_ref, bb_ref,
                   e_ref):
    t = jnp.maximum(x_ref[...] @ w1_ref[...] + b1_ref[...], 0.0)
    u = t @ w2_ref[...] + b2_ref[...]
    e_ref[...] = _ln(u, s_ref[...], bb_ref[...])


def _edge_layer_body(e_ref, gs_ref, gr_ref, w1_ref, b1_ref, w2_ref, b2_ref,
                     s_ref, bb_ref, out_ref):
    t = e_ref[...] @ w1_ref[...] + gs_ref[...] + gr_ref[...] + b1_ref[...]
    t = jnp.maximum(t, 0.0)
    m = t @ w2_ref[...] + b2_ref[...]
    out_ref[...] = e_ref[...] + _ln(m, s_ref[...], bb_ref[...])


def _node_mid_body(h_ref, a0_ref, a1_ref, wh_ref, wa_ref, b1_ref, w2_ref,
                   b2_ref, s_ref, bb_ref, w1s_ref, w1r_ref,
                   h_out, hs_out, hr_out):
    agg = a0_ref[...] + a1_ref[...]
    t = jnp.maximum(h_ref[...] @ wh_ref[...] + agg @ wa_ref[...] + b1_ref[...], 0.0)
    u = t @ w2_ref[...] + b2_ref[...]
    hn = h_ref[...] + _ln(u, s_ref[...], bb_ref[...])
    h_out[...] = hn
    hs_out[...] = hn @ w1s_ref[...]
    hr_out[...] = hn @ w1r_ref[...]


def _node_last_body(h_ref, a0_ref, a1_ref, wh_ref, wa_ref, b1_ref, w2_ref,
                    b2_ref, s_ref, bb_ref, dw1_ref, db1_ref, dw2_ref, db2_ref,
                    o_ref):
    agg = a0_ref[...] + a1_ref[...]
    t = jnp.maximum(h_ref[...] @ wh_ref[...] + agg @ wa_ref[...] + b1_ref[...], 0.0)
    u = t @ w2_ref[...] + b2_ref[...]
    hn = h_ref[...] + _ln(u, s_ref[...], bb_ref[...])
    d = jnp.maximum(hn @ dw1_ref[...] + db1_ref[...], 0.0)
    o_ref[...] = d @ dw2_ref[...] + db2_ref[...]


def _enc_node_call(x, w1, b1, w2, b2, s, bb, w1s, w1r):
    n, dn = x.shape
    lat = w1.shape[1]
    grid = (n // _BN,)
    sd = jax.ShapeDtypeStruct((n, lat), jnp.float32)
    return pl.pallas_call(
        _enc_node_body,
        grid=grid,
        in_specs=[_row_spec(_BN, dn)] + [_full_spec(a.shape)
                                         for a in (w1, b1, w2, b2, s, bb, w1s, w1r)],
        out_specs=[_row_spec(_BN, lat)] * 3,
        out_shape=[sd, sd, sd],
    )(x, w1, b1, w2, b2, s, bb, w1s, w1r)


def _enc_edge_call(x, w1, b1, w2, b2, s, bb):
    e, de = x.shape
    lat = w1.shape[1]
    grid = (e // _BE,)
    return pl.pallas_call(
        _enc_edge_body,
        grid=grid,
        in_specs=[_row_spec(_BE, de)] + [_full_spec(a.shape)
                                         for a in (w1, b1, w2, b2, s, bb)],
        out_specs=_row_spec(_BE, lat),
        out_shape=jax.ShapeDtypeStruct((e, lat), jnp.float32),
    )(x, w1, b1, w2, b2, s, bb)


def _edge_layer_call(e, gs, gr, w1, b1, w2, b2, s, bb):
    ne, lat = e.shape
    grid = (ne // _BE,)
    return pl.pallas_call(
        _edge_layer_body,
        grid=grid,
        in_specs=[_row_spec(_BE, lat)] * 3 + [_full_spec(a.shape)
                                              for a in (w1, b1, w2, b2, s, bb)],
        out_specs=_row_spec(_BE, lat),
        out_shape=jax.ShapeDtypeStruct((ne, lat), jnp.float32),
    )(e, gs, gr, w1, b1, w2, b2, s, bb)


def _node_mid_call(h, a0, a1, wh, wa, b1, w2, b2, s, bb, w1s, w1r):
    n, lat = h.shape
    grid = (n // _BN,)
    sd = jax.ShapeDtypeStruct((n, lat), jnp.float32)
    return pl.pallas_call(
        _node_mid_body,
        grid=grid,
        in_specs=[_row_spec(_BN, lat)] * 3 + [_full_spec(a.shape)
                                              for a in (wh, wa, b1, w2, b2, s, bb, w1s, w1r)],
        out_specs=[_row_spec(_BN, lat)] * 3,
        out_shape=[sd, sd, sd],
    )(h, a0, a1, wh, wa, b1, w2, b2, s, bb, w1s, w1r)


def _node_last_call(h, a0, a1, wh, wa, b1, w2, b2, s, bb, dw1, db1, dw2, db2):
    n, lat = h.shape
    grid = (n // _BN,)
    return pl.pallas_call(
        _node_last_body,
        grid=grid,
        in_specs=[_row_spec(_BN, lat)] * 3 + [_full_spec(a.shape)
                                              for a in (wh, wa, b1, w2, b2, s, bb,
                                                        dw1, db1, dw2, db2)],
        out_specs=_row_spec(_BN, lat),
        out_shape=jax.ShapeDtypeStruct((n, lat), jnp.float32),
    )(h, a0, a1, wh, wa, b1, w2, b2, s, bb, dw1, db1, dw2, db2)


@functools.lru_cache(maxsize=None)
def _gather_kernel(nt, d, nch):
    ne = _NW * nch * _CK
    mesh = plsc.VectorSubcoreMesh(core_axis_name="c", subcore_axis_name="s")

    @functools.partial(
        pl.kernel,
        out_type=jax.ShapeDtypeStruct((ne, d), jnp.float32),
        mesh=mesh,
        scratch_types=[
            pltpu.VMEM((nch, _CK), jnp.int32),
            pltpu.VMEM((_CK, d), jnp.float32),
            pltpu.SemaphoreType.DMA,
        ],
    )
    def gk(table_hbm, idx_hbm, out_hbm, idx_v, rows_v, sem):
        cid = lax.axis_index("c")
        sid = lax.axis_index("s")
        wid = sid * _NC + cid
        base = wid * (nch * _CK)
        pltpu.sync_copy(idx_hbm.at[wid], idx_v)

        def body(j, carry):
            pltpu.async_copy(table_hbm.at[idx_v.at[j]], rows_v, sem).wait()
            pltpu.sync_copy(rows_v, out_hbm.at[pl.ds(base + j * _CK, _CK)])
            return carry

        lax.fori_loop(0, nch, body, 0)

    return gk


def _sc_gather(table, idx3):
    nw, nch, ck = idx3.shape
    return _gather_kernel(table.shape[0], table.shape[1], nch)(table, idx3)


@functools.lru_cache(maxsize=None)
def _scatter_kernel(n, d, nch):
    rps = n // _NS
    mesh = plsc.VectorSubcoreMesh(core_axis_name="c", subcore_axis_name="s")

    @functools.partial(
        pl.kernel,
        out_type=jax.ShapeDtypeStruct((_NC, n, d), jnp.float32),
        mesh=mesh,
        scratch_types=[
            pltpu.VMEM((nch, _CK), jnp.int32),
            pltpu.VMEM((_CK, d), jnp.float32),
            pltpu.VMEM_SHARED((n, d), jnp.float32),
            pltpu.SemaphoreType.DMA,
        ],
    )
    def sk(vals_hbm, idx_hbm, zeros_hbm, out_hbm, idx_v, rows_v, acc, sem):
        cid = lax.axis_index("c")
        sid = lax.axis_index("s")
        wid = sid * _NC + cid
        base = wid * (nch * _CK)
        r0 = sid * rps
        pltpu.sync_copy(zeros_hbm.at[pl.ds(r0, rps)], acc.at[pl.ds(r0, rps)])
        pltpu.sync_copy(idx_hbm.at[wid], idx_v)
        plsc.subcore_barrier()

        def body(j, carry):
            pltpu.sync_copy(vals_hbm.at[pl.ds(base + j * _CK, _CK)], rows_v)
            pltpu.sync_copy(rows_v, acc.at[idx_v.at[j]], add=True)
            return carry

        lax.fori_loop(0, nch, body, 0)
        plsc.subcore_barrier()
        pltpu.sync_copy(acc.at[pl.ds(r0, rps)], out_hbm.at[cid, pl.ds(r0, rps)])

    return sk


def _sc_scatter(vals, idx3, zeros):
    nw, nch, ck = idx3.shape
    n, d = zeros.shape
    return _scatter_kernel(n, d, nch)(vals, idx3, zeros)


def kernel(node_in, edge_in, senders, receivers, params):
    p = params
    n, dn = node_in.shape
    ne, de = edge_in.shape
    lat = p['enc_n_W1'].shape[1]
    nlayers = p['msg_W1'].shape[0]
    nout = p['dec_W2'].shape[1]
    f32 = jnp.float32

    inv_n = 1.0 / p['node_norm_std']
    enc_n_W1 = p['enc_n_W1'] * inv_n[:, None]
    enc_n_b1 = p['enc_n_b1'] - (p['node_norm_mean'] * inv_n) @ p['enc_n_W1']
    inv_e = 1.0 / p['edge_norm_std']
    enc_e_W1 = p['enc_e_W1'] * inv_e[:, None]
    enc_e_b1 = p['enc_e_b1'] - (p['edge_norm_mean'] * inv_e) @ p['enc_e_W1']

    dec_W2 = p['dec_W2'] * p['out_norm_std'][None, :]
    dec_b2 = p['dec_b2'] * p['out_norm_std'] + p['out_norm_mean']
    dec_W2p = jnp.zeros((lat, lat), f32).at[:, :nout].set(dec_W2)
    dec_b2p = jnp.zeros((lat,), f32).at[:nout].set(dec_b2)

    w1e = p['msg_W1'][:, :lat]
    w1s = p['msg_W1'][:, lat:2 * lat]
    w1r = p['msg_W1'][:, 2 * lat:]
    wh = p['node_W1'][:, :lat]
    wa = p['node_W1'][:, lat:]

    def row(v):
        return v.reshape(1, -1)

    nch = ne // (_NW * _CK)
    s3 = senders.astype(jnp.int32).reshape(_NW, nch, _CK)
    r3 = receivers.astype(jnp.int32).reshape(_NW, nch, _CK)
    zeros = jnp.zeros((n, lat), f32)

    h, hs, hr = _enc_node_call(
        node_in, enc_n_W1, row(enc_n_b1), p['enc_n_W2'], row(p['enc_n_b2']),
        row(p['enc_n_ln_s']), row(p['enc_n_ln_b']), w1s[0], w1r[0])
    e = _enc_edge_call(
        edge_in, enc_e_W1, row(enc_e_b1), p['enc_e_W2'], row(p['enc_e_b2']),
        row(p['enc_e_ln_s']), row(p['enc_e_ln_b']))

    for l in range(nlayers):
        gs = _sc_gather(hs, s3)
        gr = _sc_gather(hr, r3)
        e = _edge_layer_call(
            e, gs, gr, w1e[l], row(p['msg_b1'][l]), p['msg_W2'][l],
            row(p['msg_b2'][l]), row(p['msg_ln_s'][l]), row(p['msg_ln_b'][l]))
        parts = _sc_scatter(e, r3, zeros)
        if l + 1 < nlayers:
            h, hs, hr = _node_mid_call(
                h, parts[0], parts[1], wh[l], wa[l], row(p['node_b1'][l]),
                p['node_W2'][l], row(p['node_b2'][l]), row(p['node_ln_s'][l]),
                row(p['node_ln_b'][l]), w1s[l + 1], w1r[l + 1])
        else:
            out = _node_last_call(
                h, parts[0], parts[1], wh[l], wa[l], row(p['node_b1'][l]),
                p['node_W2'][l], row(p['node_b2'][l]), row(p['node_ln_s'][l]),
                row(p['node_ln_b'][l]), p['dec_W1'], row(p['dec_b1']),
                dec_W2p, row(dec_b2p))

    return out[:, :nout]


# speedup vs baseline: 2.9807x; 2.9807x over previous
import functools

import jax
import jax.numpy as jnp
from jax import lax
from jax.experimental import pallas as pl
from jax.experimental.pallas import tpu as pltpu
from jax.experimental.pallas import tpu_sc as plsc

_NC = 2
_NS = 16
_NW = _NC * _NS
_CK = 80

_BE = 2560
_BN = 2000


def _ln(x, s, b):
    mu = jnp.mean(x, axis=-1, keepdims=True)
    xc = x - mu
    var = jnp.mean(xc * xc, axis=-1, keepdims=True)
    return xc * lax.rsqrt(var + 1e-5) * s + b


def _row_spec(br, c):
    return pl.BlockSpec((br, c), lambda i: (i, 0))


def _full_spec(shape):
    nd = len(shape)
    return pl.BlockSpec(shape, lambda i, _n=nd: (0,) * _n)


def _enc_node_body(x_ref, w1_ref, b1_ref, w2_ref, b2_ref, s_ref, bb_ref,
                   w1s_ref, w1r_ref, h_ref, hs_ref, hr_ref):
    t = jnp.maximum(x_ref[...] @ w1_ref[...] + b1_ref[...], 0.0)
    u = t @ w2_ref[...] + b2_ref[...]
    h = _ln(u, s_ref[...], bb_ref[...])
    h_ref[...] = h
    hs_ref[...] = h @ w1s_ref[...]
    hr_ref[...] = h @ w1r_ref[...]


def _enc_edge_body(x_ref, w1_ref, b1_ref, w2_ref, b2_ref, s_ref, bb_ref,
                   e_ref):
    t = jnp.maximum(x_ref[...] @ w1_ref[...] + b1_ref[...], 0.0)
    u = t @ w2_ref[...] + b2_ref[...]
    e_ref[...] = _ln(u, s_ref[...], bb_ref[...])


def _edge_layer_body(e_ref, gs_ref, gr_ref, w1_ref, b1_ref, w2_ref, b2_ref,
                     s_ref, bb_ref, out_ref):
    t = e_ref[...] @ w1_ref[...] + gs_ref[...] + gr_ref[...] + b1_ref[...]
    t = jnp.maximum(t, 0.0)
    m = t @ w2_ref[...] + b2_ref[...]
    out_ref[...] = e_ref[...] + _ln(m, s_ref[...], bb_ref[...])


def _node_mid_body(h_ref, a0_ref, a1_ref, wh_ref, wa_ref, b1_ref, w2_ref,
                   b2_ref, s_ref, bb_ref, w1s_ref, w1r_ref,
                   h_out, hs_out, hr_out):
    agg = a0_ref[...] + a1_ref[...]
    t = jnp.maximum(h_ref[...] @ wh_ref[...] + agg @ wa_ref[...] + b1_ref[...], 0.0)
    u = t @ w2_ref[...] + b2_ref[...]
    hn = h_ref[...] + _ln(u, s_ref[...], bb_ref[...])
    h_out[...] = hn
    hs_out[...] = hn @ w1s_ref[...]
    hr_out[...] = hn @ w1r_ref[...]


def _node_last_body(h_ref, a0_ref, a1_ref, wh_ref, wa_ref, b1_ref, w2_ref,
                    b2_ref, s_ref, bb_ref, dw1_ref, db1_ref, dw2_ref, db2_ref,
                    o_ref):
    agg = a0_ref[...] + a1_ref[...]
    t = jnp.maximum(h_ref[...] @ wh_ref[...] + agg @ wa_ref[...] + b1_ref[...], 0.0)
    u = t @ w2_ref[...] + b2_ref[...]
    hn = h_ref[...] + _ln(u, s_ref[...], bb_ref[...])
    d = jnp.maximum(hn @ dw1_ref[...] + db1_ref[...], 0.0)
    o_ref[...] = d @ dw2_ref[...] + db2_ref[...]


def _enc_node_call(x, w1, b1, w2, b2, s, bb, w1s, w1r):
    n, dn = x.shape
    lat = w1.shape[1]
    grid = (n // _BN,)
    sd = jax.ShapeDtypeStruct((n, lat), jnp.float32)
    return pl.pallas_call(
        _enc_node_body,
        grid=grid,
        in_specs=[_row_spec(_BN, dn)] + [_full_spec(a.shape)
                                         for a in (w1, b1, w2, b2, s, bb, w1s, w1r)],
        out_specs=[_row_spec(_BN, lat)] * 3,
        out_shape=[sd, sd, sd],
    )(x, w1, b1, w2, b2, s, bb, w1s, w1r)


def _enc_edge_call(x, w1, b1, w2, b2, s, bb):
    e, de = x.shape
    lat = w1.shape[1]
    grid = (e // _BE,)
    return pl.pallas_call(
        _enc_edge_body,
        grid=grid,
        in_specs=[_row_spec(_BE, de)] + [_full_spec(a.shape)
                                         for a in (w1, b1, w2, b2, s, bb)],
        out_specs=_row_spec(_BE, lat),
        out_shape=jax.ShapeDtypeStruct((e, lat), jnp.float32),
    )(x, w1, b1, w2, b2, s, bb)


def _edge_layer_call(e, gs, gr, w1, b1, w2, b2, s, bb):
    ne, lat = e.shape
    grid = (ne // _BE,)
    return pl.pallas_call(
        _edge_layer_body,
        grid=grid,
        in_specs=[_row_spec(_BE, lat)] * 3 + [_full_spec(a.shape)
                                              for a in (w1, b1, w2, b2, s, bb)],
        out_specs=_row_spec(_BE, lat),
        out_shape=jax.ShapeDtypeStruct((ne, lat), jnp.float32),
    )(e, gs, gr, w1, b1, w2, b2, s, bb)


def _node_mid_call(h, a0, a1, wh, wa, b1, w2, b2, s, bb, w1s, w1r):
    n, lat = h.shape
    grid = (n // _BN,)
    sd = jax.ShapeDtypeStruct((n, lat), jnp.float32)
    return pl.pallas_call(
        _node_mid_body,
        grid=grid,
        in_specs=[_row_spec(_BN, lat)] * 3 + [_full_spec(a.shape)
                                              for a in (wh, wa, b1, w2, b2, s, bb, w1s, w1r)],
        out_specs=[_row_spec(_BN, lat)] * 3,
        out_shape=[sd, sd, sd],
    )(h, a0, a1, wh, wa, b1, w2, b2, s, bb, w1s, w1r)


def _node_last_call(h, a0, a1, wh, wa, b1, w2, b2, s, bb, dw1, db1, dw2, db2):
    n, lat = h.shape
    grid = (n // _BN,)
    return pl.pallas_call(
        _node_last_body,
        grid=grid,
        in_specs=[_row_spec(_BN, lat)] * 3 + [_full_spec(a.shape)
                                              for a in (wh, wa, b1, w2, b2, s, bb,
                                                        dw1, db1, dw2, db2)],
        out_specs=_row_spec(_BN, lat),
        out_shape=jax.ShapeDtypeStruct((n, lat), jnp.float32),
    )(h, a0, a1, wh, wa, b1, w2, b2, s, bb, dw1, db1, dw2, db2)


@functools.lru_cache(maxsize=None)
def _gather_kernel(nt, d, nch):
    ne = _NW * nch * _CK
    mesh = plsc.VectorSubcoreMesh(core_axis_name="c", subcore_axis_name="s")

    @functools.partial(
        pl.kernel,
        out_type=jax.ShapeDtypeStruct((ne, d), jnp.float32),
        mesh=mesh,
        scratch_types=[
            pltpu.VMEM((nch, _CK), jnp.int32),
            pltpu.VMEM((_CK, d), jnp.float32),
            pltpu.SemaphoreType.DMA,
        ],
    )
    def gk(table_hbm, idx_hbm, out_hbm, idx_v, rows_v, sem):
        cid = lax.axis_index("c")
        sid = lax.axis_index("s")
        wid = sid * _NC + cid
        base = wid * (nch * _CK)
        pltpu.sync_copy(idx_hbm.at[wid], idx_v)

        def body(j, carry):
            pltpu.async_copy(table_hbm.at[idx_v.at[j]], rows_v, sem).wait()
            pltpu.sync_copy(rows_v, out_hbm.at[pl.ds(base + j * _CK, _CK)])
            return carry

        lax.fori_loop(0, nch, body, 0)

    return gk


def _sc_gather(table, idx3):
    nw, nch, ck = idx3.shape
    return _gather_kernel(table.shape[0], table.shape[1], nch)(table, idx3)


@functools.lru_cache(maxsize=None)
def _scatter_kernel(n, d, nch):
    rps = n // _NS
    assert rps % 8 == 0 and rps * _NS == n
    mesh = plsc.VectorSubcoreMesh(core_axis_name="c", subcore_axis_name="s")

    @functools.partial(
        pl.kernel,
        out_type=jax.ShapeDtypeStruct((_NC, n, d), jnp.float32),
        mesh=mesh,
        scratch_types=[
            pltpu.VMEM((nch, _CK), jnp.int32),
            pltpu.VMEM((_CK, d), jnp.float32),
            pltpu.VMEM_SHARED((n, d), jnp.float32),
            pltpu.SemaphoreType.DMA,
        ],
    )
    def sk(vals_hbm, idx_hbm, zeros_hbm, out_hbm, idx_v, rows_v, acc, sem):
        cid = lax.axis_index("c")
        sid = lax.axis_index("s")
        wid = sid * _NC + cid
        base = wid * (nch * _CK)
        r0 = sid * rps
        pltpu.sync_copy(zeros_hbm.at[pl.ds(r0, rps)], acc.at[pl.ds(r0, rps)])
        pltpu.sync_copy(idx_hbm.at[wid], idx_v)
        plsc.subcore_barrier()

        def body(j, carry):
            pltpu.sync_copy(vals_hbm.at[pl.ds(base + j * _CK, _CK)], rows_v)
            pltpu.sync_copy(rows_v, acc.at[idx_v.at[j]], add=True)
            return carry

        lax.fori_loop(0, nch, body, 0)
        plsc.subcore_barrier()
        pltpu.sync_copy(acc.at[pl.ds(r0, rps)], out_hbm.at[cid, pl.ds(r0, rps)])

    return sk


def _sc_scatter(vals, idx3, zeros):
    nw, nch, ck = idx3.shape
    npad, d = zeros.shape
    return _scatter_kernel(npad, d, nch)(vals, idx3, zeros)


def kernel(node_in, edge_in, senders, receivers, params):
    p = params
    n, dn = node_in.shape
    ne, de = edge_in.shape
    lat = p['enc_n_W1'].shape[1]
    nlayers = p['msg_W1'].shape[0]
    nout = p['dec_W2'].shape[1]
    f32 = jnp.float32

    inv_n = 1.0 / p['node_norm_std']
    enc_n_W1 = p['enc_n_W1'] * inv_n[:, None]
    enc_n_b1 = p['enc_n_b1'] - (p['node_norm_mean'] * inv_n) @ p['enc_n_W1']
    inv_e = 1.0 / p['edge_norm_std']
    enc_e_W1 = p['enc_e_W1'] * inv_e[:, None]
    enc_e_b1 = p['enc_e_b1'] - (p['edge_norm_mean'] * inv_e) @ p['enc_e_W1']

    dec_W2 = p['dec_W2'] * p['out_norm_std'][None, :]
    dec_b2 = p['dec_b2'] * p['out_norm_std'] + p['out_norm_mean']
    dec_W2p = jnp.zeros((lat, lat), f32).at[:, :nout].set(dec_W2)
    dec_b2p = jnp.zeros((lat,), f32).at[:nout].set(dec_b2)

    w1e = p['msg_W1'][:, :lat]
    w1s = p['msg_W1'][:, lat:2 * lat]
    w1r = p['msg_W1'][:, 2 * lat:]
    wh = p['node_W1'][:, :lat]
    wa = p['node_W1'][:, lat:]

    def row(v):
        return v.reshape(1, -1)

    nch = ne // (_NW * _CK)
    s3 = senders.astype(jnp.int32).reshape(_NW, nch, _CK)
    r3 = receivers.astype(jnp.int32).reshape(_NW, nch, _CK)
    npad = -(-n // (8 * _NS)) * (8 * _NS)
    zeros = jnp.zeros((npad, lat), f32)

    h, hs, hr = _enc_node_call(
        node_in, enc_n_W1, row(enc_n_b1), p['enc_n_W2'], row(p['enc_n_b2']),
        row(p['enc_n_ln_s']), row(p['enc_n_ln_b']), w1s[0], w1r[0])
    e = _enc_edge_call(
        edge_in, enc_e_W1, row(enc_e_b1), p['enc_e_W2'], row(p['enc_e_b2']),
        row(p['enc_e_ln_s']), row(p['enc_e_ln_b']))

    for l in range(nlayers):
        gs = _sc_gather(hs, s3)
        gr = _sc_gather(hr, r3)
        e = _edge_layer_call(
            e, gs, gr, w1e[l], row(p['msg_b1'][l]), p['msg_W2'][l],
            row(p['msg_b2'][l]), row(p['msg_ln_s'][l]), row(p['msg_ln_b'][l]))
        parts = _sc_scatter(e, r3, zeros)
        if l + 1 < nlayers:
            h, hs, hr = _node_mid_call(
                h, parts[0], parts[1], wh[l], wa[l], row(p['node_b1'][l]),
                p['node_W2'][l], row(p['node_b2'][l]), row(p['node_ln_s'][l]),
                row(p['node_ln_b'][l]), w1s[l + 1], w1r[l + 1])
        else:
            out = _node_last_call(
                h, parts[0], parts[1], wh[l], wa[l], row(p['node_b1'][l]),
                p['node_W2'][l], row(p['node_b2'][l]), row(p['node_ln_s'][l]),
                row(p['node_ln_b'][l]), p['dec_W1'], row(p['dec_b1']),
                dec_W2p, row(dec_b2p))

    return out[:, :nout]
